# baseline (speedup 1.0000x reference)
import functools

import jax
import jax.numpy as jnp
from jax import lax
from jax.experimental import pallas as pl
from jax.experimental.pallas import tpu as pltpu
from jax.experimental.pallas import tpu_sc as plsc

NC, NS, L = 2, 16, 16
NW = NC * NS


def _bracket_sc(u1w, u2w, p2, coef, segpack, masks, D, BC, NP):
    T = BC // L
    SEGN = segpack.shape[0]
    mesh = plsc.VectorSubcoreMesh(
        core_axis_name="c", subcore_axis_name="s",
        num_cores=NC, num_subcores=NS)

    @functools.partial(
        pl.kernel,
        out_type=jax.ShapeDtypeStruct((NW, D * BC), jnp.float32),
        mesh=mesh,
        compiler_params=pltpu.CompilerParams(needs_layout_passes=False),
        scratch_types=[
            pltpu.VMEM((D * BC,), jnp.float32),
            pltpu.VMEM((D * BC,), jnp.float32),
            pltpu.VMEM((D * BC,), jnp.float32),
            pltpu.VMEM((NP,), jnp.int32),
            pltpu.VMEM((NP,), jnp.float32),
            pltpu.VMEM((SEGN,), jnp.int32),
            pltpu.VMEM((17 * L,), jnp.float32),
        ],
    )
    def sc_kernel(u1_hbm, u2_hbm, p_hbm, c_hbm, seg_hbm, mask_hbm, out_hbm,
                  u1_v, u2_v, o_v, p_v, c_v, seg_v, mask_v):
        wid = lax.axis_index("s") * NC + lax.axis_index("c")
        pltpu.sync_copy(u1_hbm.at[wid], u1_v)
        pltpu.sync_copy(u2_hbm.at[wid], u2_v)
        pltpu.sync_copy(p_hbm, p_v)
        pltpu.sync_copy(c_hbm, c_v)
        pltpu.sync_copy(seg_hbm, seg_v)
        pltpu.sync_copy(mask_hbm, mask_v)

        zvec = jnp.zeros((L,), jnp.float32)
        iota = lax.iota(jnp.int32, L)
        zeros = tuple(zvec for _ in range(T))

        def seg_body(d, carry):
            pk = seg_v[pl.ds(d, L)][0]
            start = lax.shift_right_logical(pk, 14)
            ln = jnp.bitwise_and(pk, 16383)
            nch = lax.shift_right_logical(ln + 15, 4)

            def chunk_body(q, acc):
                n0 = start + q * L
                pv = p_v[pl.ds(n0, L)]
                cv = c_v[pl.ds(n0, L)]
                msel = jnp.minimum(ln - q * L, L)
                cv = cv * mask_v[pl.ds(msel * L, L)]
                ivO = lax.shift_right_logical(pv, 9) * BC
                jvO = jnp.bitwise_and(pv, 511) * BC
                for m in range(L):
                    ivec = jnp.broadcast_to(ivO[m], (L,)) + iota
                    jvec = jnp.broadcast_to(jvO[m], (L,)) + iota
                    cvec = jnp.broadcast_to(cv[m], (L,))
                    new_acc = []
                    for t in range(T):
                        a = plsc.load_gather(
                            u1_v.at[pl.ds(t * L, D * BC - t * L)], [ivec])
                        b = plsc.load_gather(
                            u2_v.at[pl.ds(t * L, D * BC - t * L)], [jvec])
                        new_acc.append(acc[t] + a * b * cvec)
                    acc = tuple(new_acc)
                return acc

            accf = lax.fori_loop(0, nch, chunk_body, zeros)
            for t in range(T):
                o_v[pl.ds(d * BC + t * L, L)] = accf[t]
            return carry

        lax.fori_loop(0, D, seg_body, 0)

        pltpu.sync_copy(o_v, out_hbm.at[wid])

    return sc_kernel(u1w, u2w, p2, coef, segpack, masks)


def _sym_scalar_tc(v1, v2):
    B, D = v1.shape
    blk = 256

    def body(v1_ref, v2_ref, sym_ref, sc_ref):
        p = v1_ref[...] * v2_ref[...]
        sym_ref[...] = p
        sc_ref[...] = jnp.sum(p, axis=-1, keepdims=True)

    return pl.pallas_call(
        body,
        grid=(B // blk,),
        in_specs=[
            pl.BlockSpec((blk, D), lambda b: (b, 0)),
            pl.BlockSpec((blk, D), lambda b: (b, 0)),
        ],
        out_specs=[
            pl.BlockSpec((blk, D), lambda b: (b, 0)),
            pl.BlockSpec((blk, 1), lambda b: (b, 0)),
        ],
        out_shape=[
            jax.ShapeDtypeStruct((B, D), jnp.float32),
            jax.ShapeDtypeStruct((B, 1), jnp.float32),
        ],
    )(v1, v2)


def kernel(v1, v2, I, J, K, C):
    B, D = v1.shape
    NNZ = I.shape[0]
    BC = B // NW

    I = I.astype(jnp.int32)
    J = J.astype(jnp.int32)
    K = K.astype(jnp.int32)
    C = C.astype(jnp.float32)

    P2 = I * 512 + J
    P2 = jnp.concatenate([P2, jnp.zeros((L,), jnp.int32)])
    Cp = jnp.concatenate([C, jnp.zeros((L,), jnp.float32)])
    NP = NNZ + L

    seg_bnd = jnp.sum(
        K[None, :] < jnp.arange(D + 1, dtype=jnp.int32)[:, None],
        axis=1).astype(jnp.int32)
    seg_start = seg_bnd[:D]
    lens = seg_bnd[1:] - seg_start
    segpack = seg_start * 16384 + lens
    segpack = jnp.concatenate([segpack, jnp.zeros((L,), jnp.int32)])

    masks = (jnp.arange(L, dtype=jnp.int32)[None, :]
             < jnp.arange(17, dtype=jnp.int32)[:, None]
             ).astype(jnp.float32).reshape(-1)

    u1w = jnp.transpose(v1.reshape(NW, BC, D), (0, 2, 1)).reshape(NW, D * BC)
    u2w = jnp.transpose(v2.reshape(NW, BC, D), (0, 2, 1)).reshape(NW, D * BC)
    outw = _bracket_sc(u1w, u2w, P2, Cp, segpack, masks, D, BC, NP)
    antisym = jnp.transpose(outw.reshape(NW, D, BC), (0, 2, 1)).reshape(B, D)

    sym, scalar = _sym_scalar_tc(v1, v2)
    return (antisym, sym, scalar)


# speedup vs baseline: 2.1135x; 2.1135x over previous
import functools

import jax
import jax.numpy as jnp
from jax import lax
from jax.experimental import pallas as pl
from jax.experimental.pallas import tpu as pltpu
from jax.experimental.pallas import tpu_sc as plsc

NC, NS, L = 2, 16, 16
NW = NC * NS


def _bracket_sc(u1w, u2w, p3, coef, segpack, D, BC, NP, S):
    T = BC // L
    SEGN = segpack.shape[0]
    FD = ((D + 2 * L - 1) // L) * L
    mesh = plsc.VectorSubcoreMesh(
        core_axis_name="c", subcore_axis_name="s",
        num_cores=NC, num_subcores=NS)

    @functools.partial(
        pl.kernel,
        out_type=jax.ShapeDtypeStruct((NW, D * BC), jnp.float32),
        mesh=mesh,
        compiler_params=pltpu.CompilerParams(needs_layout_passes=False),
        scratch_types=[
            pltpu.VMEM((D * BC,), jnp.float32),
            pltpu.VMEM((D * BC,), jnp.float32),
            pltpu.VMEM((D * BC,), jnp.float32),
            pltpu.VMEM((FD,), jnp.float32),
            pltpu.VMEM((NP,), jnp.int32),
            pltpu.VMEM((NP,), jnp.float32),
            pltpu.VMEM((S,), jnp.int32),
            pltpu.VMEM((S,), jnp.float32),
            pltpu.VMEM((SEGN,), jnp.int32),
        ],
    )
    def sc_kernel(u1_hbm, u2_hbm, p_hbm, c_hbm, seg_hbm, out_hbm,
                  u1_v, u2_v, o_v, f_v, p_v, c_v, pp_v, cp_v, seg_v):
        wid = lax.axis_index("s") * NC + lax.axis_index("c")
        pltpu.sync_copy(u1_hbm.at[wid], u1_v)
        pltpu.sync_copy(u2_hbm.at[wid], u2_v)
        pltpu.sync_copy(p_hbm, p_v)
        pltpu.sync_copy(c_hbm, c_v)
        pltpu.sync_copy(seg_hbm, seg_v)

        zvec = jnp.zeros((L,), jnp.float32)
        ones = jnp.ones((L,), jnp.float32)
        iota = lax.iota(jnp.int32, L)
        kd1 = jnp.full((L,), D - 1, jnp.int32)
        zeros = tuple(zvec for _ in range(T))

        def zero_body(d, carry):
            for t in range(T):
                o_v[pl.ds(d * BC + t * L, L)] = zvec
            return carry

        lax.fori_loop(0, D, zero_body, 0)

        def zero_flags(q, carry):
            f_v[pl.ds(q * L, L)] = zvec
            return carry

        lax.fori_loop(0, FD // L, zero_flags, 0)

        def prefill(q, carry):
            pp_v[pl.ds(q * L, L)] = kd1
            cp_v[pl.ds(q * L, L)] = zvec
            return carry

        lax.fori_loop(0, S // L, prefill, 0)

        def build_seg(d, cb):
            pk = seg_v[pl.ds(d, L)][0]
            start = lax.shift_right_logical(pk, 14)
            ln = jnp.bitwise_and(pk, 16383)
            nch = lax.shift_right_logical(ln + 15, 4)
            lnv = jnp.broadcast_to(ln, (L,)) - iota

            def build_chunk(q, carry):
                src = start + q * L
                dst = cb + q * L
                mm = jnp.minimum(jnp.maximum(lnv - q * L, 0), 1)
                pp_v[pl.ds(dst, L)] = p_v[pl.ds(src, L)]
                cp_v[pl.ds(dst, L)] = c_v[pl.ds(src, L)] * mm.astype(jnp.float32)
                return carry

            lax.fori_loop(0, nch, build_chunk, 0)
            return cb + nch * L

        lax.fori_loop(0, D, build_seg, jnp.int32(0))

        def body(g, acc):
            base = g * L
            pv = pp_v[pl.ds(base, L)]
            cv = cp_v[pl.ds(base, L)]
            ivO = lax.shift_right_logical(pv, 18) * BC
            jvO = jnp.bitwise_and(lax.shift_right_logical(pv, 9), 511) * BC
            kv = jnp.bitwise_and(pv, 511)
            for m in range(L):
                ivec = jnp.broadcast_to(ivO[m], (L,)) + iota
                jvec = jnp.broadcast_to(jvO[m], (L,)) + iota
                cvec = jnp.broadcast_to(cv[m], (L,))
                new_acc = []
                for t in range(T):
                    a = plsc.load_gather(
                        u1_v.at[pl.ds(t * L, D * BC - t * L)], [ivec])
                    b = plsc.load_gather(
                        u2_v.at[pl.ds(t * L, D * BC - t * L)], [jvec])
                    new_acc.append(acc[t] + a * b * cvec)
                acc = tuple(new_acc)
            kvO = kv * BC
            kvec = jnp.broadcast_to(kvO[0], (L,)) + iota
            for t in range(T):
                plsc.store_scatter(
                    o_v.at[pl.ds(t * L, D * BC - t * L)], [kvec], acc[t])
            plsc.store_scatter(f_v, [kv], ones)
            return acc

        lax.fori_loop(0, S // L, body, zeros)

        def diff_body(d, last):
            fv = jnp.broadcast_to(f_v[pl.ds(d, L)][0], (L,))
            new_last = []
            for t in range(T):
                tmp = o_v[pl.ds(d * BC + t * L, L)]
                diff = (tmp - last[t]) * fv
                o_v[pl.ds(d * BC + t * L, L)] = diff
                new_last.append(last[t] + diff)
            return tuple(new_last)

        lax.fori_loop(0, D, diff_body, zeros)

        pltpu.sync_copy(o_v, out_hbm.at[wid])

    return sc_kernel(u1w, u2w, p3, coef, segpack)


def _sym_scalar_tc(v1, v2):
    B, D = v1.shape
    blk = 256

    def body(v1_ref, v2_ref, sym_ref, sc_ref):
        p = v1_ref[...] * v2_ref[...]
        sym_ref[...] = p
        sc_ref[...] = jnp.sum(p, axis=-1, keepdims=True)

    return pl.pallas_call(
        body,
        grid=(B // blk,),
        in_specs=[
            pl.BlockSpec((blk, D), lambda b: (b, 0)),
            pl.BlockSpec((blk, D), lambda b: (b, 0)),
        ],
        out_specs=[
            pl.BlockSpec((blk, D), lambda b: (b, 0)),
            pl.BlockSpec((blk, 1), lambda b: (b, 0)),
        ],
        out_shape=[
            jax.ShapeDtypeStruct((B, D), jnp.float32),
            jax.ShapeDtypeStruct((B, 1), jnp.float32),
        ],
    )(v1, v2)


def kernel(v1, v2, I, J, K, C):
    B, D = v1.shape
    NNZ = I.shape[0]
    BC = B // NW

    I = I.astype(jnp.int32)
    J = J.astype(jnp.int32)
    K = K.astype(jnp.int32)
    C = C.astype(jnp.float32)

    P3 = (I * 512 + J) * 512 + K
    P3 = jnp.concatenate([P3, jnp.full((L,), D - 1, jnp.int32)])
    Cp = jnp.concatenate([C, jnp.zeros((L,), jnp.float32)])
    NP = NNZ + L

    seg_bnd = jnp.sum(
        K[None, :] < jnp.arange(D + 1, dtype=jnp.int32)[:, None],
        axis=1).astype(jnp.int32)
    seg_start = seg_bnd[:D]
    lens = seg_bnd[1:] - seg_start
    segpack = seg_start * 16384 + lens
    segpack = jnp.concatenate([segpack, jnp.zeros((L,), jnp.int32)])

    S = ((NNZ + (L - 1) * D + L - 1) // L) * L

    u1w = jnp.transpose(v1.reshape(NW, BC, D), (0, 2, 1)).reshape(NW, D * BC)
    u2w = jnp.transpose(v2.reshape(NW, BC, D), (0, 2, 1)).reshape(NW, D * BC)
    outw = _bracket_sc(u1w, u2w, P3, Cp, segpack, D, BC, NP, S)
    antisym = jnp.transpose(outw.reshape(NW, D, BC), (0, 2, 1)).reshape(B, D)

    sym, scalar = _sym_scalar_tc(v1, v2)
    return (antisym, sym, scalar)
